# baseline (speedup 1.0000x reference)
import functools

import jax
import jax.numpy as jnp
from jax import lax
from jax.experimental import pallas as pl
from jax.experimental.pallas import tpu as pltpu
from jax.experimental.pallas import tpu_sc as plsc

_NUM_CORES = 2
_NUM_SUBCORES = 16
_NW = _NUM_CORES * _NUM_SUBCORES

_CHUNK = 256
_NBUF = 5

_V = 1000000
_D = 64
_LANES = 128
_NCOLS_MAIN = _V // _LANES
_V_TAIL = _V - _NCOLS_MAIN * _LANES


def _mesh():
    return plsc.VectorSubcoreMesh(
        core_axis_name="c",
        subcore_axis_name="s",
        num_cores=_NUM_CORES,
        num_subcores=_NUM_SUBCORES,
    )


def _transpose_block(src, dst, n_pair_rows):
    iota = lax.iota(jnp.int32, 16)

    @pl.loop(0, n_pair_rows)
    def _(j):
        for k in range(8):
            row_vec = iota + ((16 * k) & 63)
            col_vec = jnp.full((16,), 2 * j + (k >> 2), jnp.int32)
            vals = plsc.load_gather(src, [row_vec, col_vec])
            dst[j, pl.ds(16 * k, 16)] = vals


@functools.partial(
    pl.kernel,
    out_type=jax.ShapeDtypeStruct((_V // 2, _LANES), jnp.float32),
    mesh=_mesh(),
    scratch_types=[
        pltpu.VMEM((2, _D, _LANES), jnp.float32),
        pltpu.VMEM((2, _D, _LANES), jnp.float32),
        pltpu.SemaphoreType.DMA,
        pltpu.SemaphoreType.DMA,
        pltpu.SemaphoreType.DMA,
        pltpu.SemaphoreType.DMA,
    ],
    compiler_params=pltpu.CompilerParams(
        use_tc_tiling_on_sc=True,
        needs_layout_passes=False,
        disable_bounds_checks=True,
    ),
    name="emb_relayout",
)
def _relayout(tt_hbm, out_hbm, in_v, tp_v, isem0, isem1, osem0, osem1):
    isems = (isem0, isem1)
    osems = (osem0, osem1)
    wid = lax.axis_index("s") * _NUM_CORES + lax.axis_index("c")
    ncols = _NCOLS_MAIN + 1
    n_extra = ncols - 244 * _NW
    tail_wid = (ncols - 1) % _NW
    nc = 244 + jnp.where(wid < n_extra, 1, 0)

    def col(t):
        return wid + _NW * t

    def start_in(t, b):
        pltpu.async_copy(
            tt_hbm.at[:, pl.ds(col(t) * _LANES, _LANES)], in_v.at[b], isems[b]
        )

    def wait_in(b):
        pltpu.make_async_copy(
            tt_hbm.at[:, pl.ds(0, _LANES)], in_v.at[b], isems[b]
        ).wait()

    def start_out(t, b):
        c = col(t)

        @pl.when(c != _NCOLS_MAIN)
        def _():
            pltpu.async_copy(
                tp_v.at[b], out_hbm.at[pl.ds(c * (_LANES // 2), _D)], osems[b]
            )

        @pl.when(c == _NCOLS_MAIN)
        def _():
            pltpu.async_copy(
                tp_v.at[b, pl.ds(0, _D // 2)],
                out_hbm.at[pl.ds(c * (_LANES // 2), _D // 2)],
                osems[b],
            )

    def wait_out(b):
        pltpu.make_async_copy(
            tt_hbm.at[pl.ds(0, _D), pl.ds(0, _LANES)], tp_v.at[b], osems[b]
        ).wait()

    def wait_out_half(b):
        pltpu.make_async_copy(
            tt_hbm.at[pl.ds(0, _D // 2), pl.ds(0, _LANES)],
            tp_v.at[b, pl.ds(0, _D // 2)],
            osems[b],
        ).wait()

    start_in(0, 0)

    @pl.loop(0, (nc + 1) // 2 * 2, step=2)
    def _(t0):
        for b in range(2):
            t = t0 + b

            @pl.when(t < nc)
            def _():
                @pl.when(t + 1 < nc)
                def _():
                    start_in(t + 1, 1 - b)

                wait_in(b)

                @pl.when(t >= 2)
                def _():
                    wait_out(b)

                _transpose_block(in_v.at[b], tp_v.at[b], _D)
                start_out(t, b)

    @pl.when(wid == tail_wid)
    def _():
        wait_out_half(0)
        wait_out(1)

    @pl.when(wid != tail_wid)
    def _():
        wait_out(0)
        wait_out(1)


@functools.lru_cache(maxsize=None)
def _build_gather(B, V, D):
    assert B % _NW == 0
    b_per_w = B // _NW
    assert b_per_w % _CHUNK == 0
    n_chunks = b_per_w // _CHUNK
    assert n_chunks % _NBUF == 0

    @functools.partial(
        pl.kernel,
        out_type=jax.ShapeDtypeStruct((B, D), jnp.float32),
        mesh=_mesh(),
        scratch_types=[
            pltpu.VMEM((b_per_w,), jnp.int32),
            pltpu.VMEM((_NBUF, _CHUNK, D), jnp.float32),
        ]
        + [pltpu.SemaphoreType.DMA] * _NBUF,
        compiler_params=pltpu.CompilerParams(use_tc_tiling_on_sc=False),
        name="emb_gather",
    )
    def emb_gather(table_hbm, idx_hbm, out_hbm, idx_v, rows_v, *sems):
        wid = lax.axis_index("s") * _NUM_CORES + lax.axis_index("c")
        base = wid * b_per_w

        pltpu.sync_copy(idx_hbm.at[pl.ds(base, b_per_w)], idx_v)

        def start(g, b):
            pltpu.async_copy(
                table_hbm.at[idx_v.at[pl.ds(g * _CHUNK, _CHUNK)]],
                rows_v.at[b],
                sems[b],
            )

        def wait(b):
            pltpu.make_async_copy(
                table_hbm.at[pl.ds(0, _CHUNK)], rows_v.at[b], sems[b]
            ).wait()

        def writeout(g, b):
            pltpu.sync_copy(
                rows_v.at[b], out_hbm.at[pl.ds(base + g * _CHUNK, _CHUNK)]
            )

        for b in range(_NBUF - 1):
            start(b, b)

        @pl.loop(0, n_chunks, step=_NBUF)
        def _(g):
            for b in range(_NBUF):
                nxt = g + b + _NBUF - 1

                @pl.when(nxt < n_chunks)
                def _():
                    start(nxt, (b + _NBUF - 1) % _NBUF)

                wait(b)
                writeout(g + b, b)

    return emb_gather


def kernel(inputs, embeddings):
    V, D = embeddings.shape
    B = inputs.shape[0] * inputs.shape[1]
    idx = inputs.reshape(-1).astype(jnp.int32)
    linear = _relayout(embeddings.T).reshape(V, D)
    out = _build_gather(B, V, D)(linear, idx)
    return out.reshape(inputs.shape + (D,))


# speedup vs baseline: 1.2036x; 1.2036x over previous
import functools

import jax
import jax.numpy as jnp
from jax import lax
from jax.experimental import pallas as pl
from jax.experimental.pallas import tpu as pltpu
from jax.experimental.pallas import tpu_sc as plsc

_NUM_CORES = 2
_NUM_SUBCORES = 16
_NW = _NUM_CORES * _NUM_SUBCORES

_CHUNK = 256
_NBUF = 5

_V = 1000000
_D = 64
_LANES = 128
_NCOLS_MAIN = _V // _LANES
_V_TAIL = _V - _NCOLS_MAIN * _LANES


def _mesh():
    return plsc.VectorSubcoreMesh(
        core_axis_name="c",
        subcore_axis_name="s",
        num_cores=_NUM_CORES,
        num_subcores=_NUM_SUBCORES,
    )


def _transpose_block(src, dst):
    iota = lax.iota(jnp.int32, 16)
    rows = [(16 * k + iota) >> 1 for k in range(8)]
    colp = [((16 * k + iota) & 1) * 64 for k in range(8)]

    @pl.loop(0, _D, unroll=4)
    def _(f):
        fb = jnp.full((16,), f, jnp.int32)
        for k in range(8):
            vals = src[f, pl.ds(16 * k, 16)]
            plsc.store_scatter(dst, [rows[k], colp[k] + fb], vals)


@functools.partial(
    pl.kernel,
    out_type=jax.ShapeDtypeStruct((_V // 2, _LANES), jnp.float32),
    mesh=_mesh(),
    scratch_types=[
        pltpu.VMEM((2, _D, _LANES), jnp.float32),
        pltpu.VMEM((2, _D, _LANES), jnp.float32),
        pltpu.SemaphoreType.DMA,
        pltpu.SemaphoreType.DMA,
        pltpu.SemaphoreType.DMA,
        pltpu.SemaphoreType.DMA,
    ],
    compiler_params=pltpu.CompilerParams(
        use_tc_tiling_on_sc=True,
        needs_layout_passes=False,
        disable_bounds_checks=True,
    ),
    name="emb_relayout",
)
def _relayout(tt_hbm, out_hbm, in_v, tp_v, isem0, isem1, osem0, osem1):
    isems = (isem0, isem1)
    osems = (osem0, osem1)
    wid = lax.axis_index("s") * _NUM_CORES + lax.axis_index("c")
    ncols = _NCOLS_MAIN + 1
    n_extra = ncols - 244 * _NW
    tail_wid = (ncols - 1) % _NW
    nc = 244 + jnp.where(wid < n_extra, 1, 0)

    def col(t):
        return wid + _NW * t

    def start_in(t, b):
        pltpu.async_copy(
            tt_hbm.at[:, pl.ds(col(t) * _LANES, _LANES)], in_v.at[b], isems[b]
        )

    def wait_in(b):
        pltpu.make_async_copy(
            tt_hbm.at[:, pl.ds(0, _LANES)], in_v.at[b], isems[b]
        ).wait()

    def start_out(t, b):
        c = col(t)

        @pl.when(c != _NCOLS_MAIN)
        def _():
            pltpu.async_copy(
                tp_v.at[b], out_hbm.at[pl.ds(c * (_LANES // 2), _D)], osems[b]
            )

        @pl.when(c == _NCOLS_MAIN)
        def _():
            pltpu.async_copy(
                tp_v.at[b, pl.ds(0, _D // 2)],
                out_hbm.at[pl.ds(c * (_LANES // 2), _D // 2)],
                osems[b],
            )

    def wait_out(b):
        pltpu.make_async_copy(
            tt_hbm.at[pl.ds(0, _D), pl.ds(0, _LANES)], tp_v.at[b], osems[b]
        ).wait()

    def wait_out_half(b):
        pltpu.make_async_copy(
            tt_hbm.at[pl.ds(0, _D // 2), pl.ds(0, _LANES)],
            tp_v.at[b, pl.ds(0, _D // 2)],
            osems[b],
        ).wait()

    start_in(0, 0)

    @pl.loop(0, (nc + 1) // 2 * 2, step=2)
    def _(t0):
        for b in range(2):
            t = t0 + b

            @pl.when(t < nc)
            def _():
                @pl.when(t + 1 < nc)
                def _():
                    start_in(t + 1, 1 - b)

                wait_in(b)

                @pl.when(t >= 2)
                def _():
                    wait_out(b)

                _transpose_block(in_v.at[b], tp_v.at[b])
                start_out(t, b)

    @pl.when(wid == tail_wid)
    def _():
        wait_out_half(0)
        wait_out(1)

    @pl.when(wid != tail_wid)
    def _():
        wait_out(0)
        wait_out(1)


@functools.lru_cache(maxsize=None)
def _build_gather(B, V, D):
    assert B % _NW == 0
    b_per_w = B // _NW
    assert b_per_w % _CHUNK == 0
    n_chunks = b_per_w // _CHUNK
    assert n_chunks % _NBUF == 0

    @functools.partial(
        pl.kernel,
        out_type=jax.ShapeDtypeStruct((B, D), jnp.float32),
        mesh=_mesh(),
        scratch_types=[
            pltpu.VMEM((b_per_w,), jnp.int32),
            pltpu.VMEM((_NBUF, _CHUNK, D), jnp.float32),
        ]
        + [pltpu.SemaphoreType.DMA] * _NBUF,
        compiler_params=pltpu.CompilerParams(use_tc_tiling_on_sc=False),
        name="emb_gather",
    )
    def emb_gather(table_hbm, idx_hbm, out_hbm, idx_v, rows_v, *sems):
        wid = lax.axis_index("s") * _NUM_CORES + lax.axis_index("c")
        base = wid * b_per_w

        pltpu.sync_copy(idx_hbm.at[pl.ds(base, b_per_w)], idx_v)

        def start(g, b):
            pltpu.async_copy(
                table_hbm.at[idx_v.at[pl.ds(g * _CHUNK, _CHUNK)]],
                rows_v.at[b],
                sems[b],
            )

        def wait(b):
            pltpu.make_async_copy(
                table_hbm.at[pl.ds(0, _CHUNK)], rows_v.at[b], sems[b]
            ).wait()

        def writeout(g, b):
            pltpu.sync_copy(
                rows_v.at[b], out_hbm.at[pl.ds(base + g * _CHUNK, _CHUNK)]
            )

        for b in range(_NBUF - 1):
            start(b, b)

        @pl.loop(0, n_chunks, step=_NBUF)
        def _(g):
            for b in range(_NBUF):
                nxt = g + b + _NBUF - 1

                @pl.when(nxt < n_chunks)
                def _():
                    start(nxt, (b + _NBUF - 1) % _NBUF)

                wait(b)
                writeout(g + b, b)

    return emb_gather


def kernel(inputs, embeddings):
    V, D = embeddings.shape
    B = inputs.shape[0] * inputs.shape[1]
    idx = inputs.reshape(-1).astype(jnp.int32)
    linear = _relayout(embeddings.T).reshape(V, D)
    out = _build_gather(B, V, D)(linear, idx)
    return out.reshape(inputs.shape + (D,))


# speedup vs baseline: 1.6108x; 1.3384x over previous
import functools

import jax
import jax.numpy as jnp
from jax import lax
from jax.experimental import pallas as pl
from jax.experimental.pallas import tpu as pltpu
from jax.experimental.pallas import tpu_sc as plsc

_NUM_CORES = 2
_NUM_SUBCORES = 16
_NW = _NUM_CORES * _NUM_SUBCORES

_CHUNK = 256
_NBUF = 5

_V = 1000000
_D = 64
_LANES = 128
_NCOLS_MAIN = _V // _LANES
_V_TAIL = _V - _NCOLS_MAIN * _LANES


def _mesh():
    return plsc.VectorSubcoreMesh(
        core_axis_name="c",
        subcore_axis_name="s",
        num_cores=_NUM_CORES,
        num_subcores=_NUM_SUBCORES,
    )


def _transpose_block(src, dst):
    iota = lax.iota(jnp.int32, 16)
    rows = [(16 * k + iota) >> 1 for k in range(8)]
    colp = [((16 * k + iota) & 1) * 64 for k in range(8)]

    @plsc.parallel_loop(0, _D, unroll=4)
    def _(f):
        fb = jnp.full((16,), f, jnp.int32)
        for k in range(8):
            vals = src[f, pl.ds(16 * k, 16)]
            plsc.store_scatter(dst, [rows[k], colp[k] + fb], vals)


@functools.partial(
    pl.kernel,
    out_type=jax.ShapeDtypeStruct((_V // 2, _LANES), jnp.float32),
    mesh=_mesh(),
    scratch_types=[
        pltpu.VMEM((2, _D, _LANES), jnp.float32),
        pltpu.VMEM((2, _D, _LANES), jnp.float32),
        pltpu.SemaphoreType.DMA,
        pltpu.SemaphoreType.DMA,
        pltpu.SemaphoreType.DMA,
        pltpu.SemaphoreType.DMA,
    ],
    compiler_params=pltpu.CompilerParams(
        use_tc_tiling_on_sc=True,
        needs_layout_passes=False,
        disable_bounds_checks=True,
    ),
    name="emb_relayout",
)
def _relayout(tt_hbm, out_hbm, in_v, tp_v, isem0, isem1, osem0, osem1):
    isems = (isem0, isem1)
    osems = (osem0, osem1)
    wid = lax.axis_index("s") * _NUM_CORES + lax.axis_index("c")
    ncols = _NCOLS_MAIN + 1
    n_extra = ncols - 244 * _NW
    tail_wid = (ncols - 1) % _NW
    nc = 244 + jnp.where(wid < n_extra, 1, 0)

    def col(t):
        return wid + _NW * t

    def start_in(t, b):
        pltpu.async_copy(
            tt_hbm.at[:, pl.ds(col(t) * _LANES, _LANES)], in_v.at[b], isems[b]
        )

    def wait_in(b):
        pltpu.make_async_copy(
            tt_hbm.at[:, pl.ds(0, _LANES)], in_v.at[b], isems[b]
        ).wait()

    def start_out(t, b):
        c = col(t)

        @pl.when(c != _NCOLS_MAIN)
        def _():
            pltpu.async_copy(
                tp_v.at[b], out_hbm.at[pl.ds(c * (_LANES // 2), _D)], osems[b]
            )

        @pl.when(c == _NCOLS_MAIN)
        def _():
            pltpu.async_copy(
                tp_v.at[b, pl.ds(0, _D // 2)],
                out_hbm.at[pl.ds(c * (_LANES // 2), _D // 2)],
                osems[b],
            )

    def wait_out(b):
        pltpu.make_async_copy(
            tt_hbm.at[pl.ds(0, _D), pl.ds(0, _LANES)], tp_v.at[b], osems[b]
        ).wait()

    def wait_out_half(b):
        pltpu.make_async_copy(
            tt_hbm.at[pl.ds(0, _D // 2), pl.ds(0, _LANES)],
            tp_v.at[b, pl.ds(0, _D // 2)],
            osems[b],
        ).wait()

    start_in(0, 0)

    @pl.loop(0, (nc + 1) // 2 * 2, step=2)
    def _(t0):
        for b in range(2):
            t = t0 + b

            @pl.when(t < nc)
            def _():
                @pl.when(t + 1 < nc)
                def _():
                    start_in(t + 1, 1 - b)

                wait_in(b)

                @pl.when(t >= 2)
                def _():
                    wait_out(b)

                _transpose_block(in_v.at[b], tp_v.at[b])
                start_out(t, b)

    @pl.when(wid == tail_wid)
    def _():
        wait_out_half(0)
        wait_out(1)

    @pl.when(wid != tail_wid)
    def _():
        wait_out(0)
        wait_out(1)


@functools.lru_cache(maxsize=None)
def _build_gather(B, V, D):
    assert B % _NW == 0
    b_per_w = B // _NW
    assert b_per_w % _CHUNK == 0
    n_chunks = b_per_w // _CHUNK
    assert n_chunks % _NBUF == 0

    @functools.partial(
        pl.kernel,
        out_type=jax.ShapeDtypeStruct((B, D), jnp.float32),
        mesh=_mesh(),
        scratch_types=[
            pltpu.VMEM((b_per_w,), jnp.int32),
            pltpu.VMEM((_NBUF, _CHUNK, D), jnp.float32),
        ]
        + [pltpu.SemaphoreType.DMA] * _NBUF,
        compiler_params=pltpu.CompilerParams(use_tc_tiling_on_sc=False),
        name="emb_gather",
    )
    def emb_gather(table_hbm, idx_hbm, out_hbm, idx_v, rows_v, *sems):
        wid = lax.axis_index("s") * _NUM_CORES + lax.axis_index("c")
        base = wid * b_per_w

        pltpu.sync_copy(idx_hbm.at[pl.ds(base, b_per_w)], idx_v)

        def start(g, b):
            pltpu.async_copy(
                table_hbm.at[idx_v.at[pl.ds(g * _CHUNK, _CHUNK)]],
                rows_v.at[b],
                sems[b],
            )

        def wait(b):
            pltpu.make_async_copy(
                table_hbm.at[pl.ds(0, _CHUNK)], rows_v.at[b], sems[b]
            ).wait()

        def writeout(g, b):
            pltpu.sync_copy(
                rows_v.at[b], out_hbm.at[pl.ds(base + g * _CHUNK, _CHUNK)]
            )

        for b in range(_NBUF - 1):
            start(b, b)

        @pl.loop(0, n_chunks, step=_NBUF)
        def _(g):
            for b in range(_NBUF):
                nxt = g + b + _NBUF - 1

                @pl.when(nxt < n_chunks)
                def _():
                    start(nxt, (b + _NBUF - 1) % _NBUF)

                wait(b)
                writeout(g + b, b)

    return emb_gather


def kernel(inputs, embeddings):
    V, D = embeddings.shape
    B = inputs.shape[0] * inputs.shape[1]
    idx = inputs.reshape(-1).astype(jnp.int32)
    linear = _relayout(embeddings.T).reshape(V, D)
    out = _build_gather(B, V, D)(linear, idx)
    return out.reshape(inputs.shape + (D,))


# speedup vs baseline: 1.6123x; 1.0009x over previous
import functools

import jax
import jax.numpy as jnp
from jax import lax
from jax.experimental import pallas as pl
from jax.experimental.pallas import tpu as pltpu
from jax.experimental.pallas import tpu_sc as plsc

_NUM_CORES = 2
_NUM_SUBCORES = 16
_NW = _NUM_CORES * _NUM_SUBCORES

_CHUNK = 256
_NBUF = 5

_V = 1000000
_D = 64
_LANES = 128
_NCOLS_MAIN = _V // _LANES
_V_TAIL = _V - _NCOLS_MAIN * _LANES


def _mesh():
    return plsc.VectorSubcoreMesh(
        core_axis_name="c",
        subcore_axis_name="s",
        num_cores=_NUM_CORES,
        num_subcores=_NUM_SUBCORES,
    )


def _transpose_block(src, dst):
    iota = lax.iota(jnp.int32, 16)
    rows = [(16 * k + iota) >> 1 for k in range(8)]
    colp = [((16 * k + iota) & 1) * 64 for k in range(8)]

    @plsc.parallel_loop(0, _D, unroll=4)
    def _(f):
        fb = jnp.full((16,), f, jnp.int32)
        for k in range(8):
            vals = src[f, pl.ds(16 * k, 16)]
            plsc.store_scatter(dst, [rows[k], colp[k] + fb], vals)


@functools.partial(
    pl.kernel,
    out_type=jax.ShapeDtypeStruct((_V // 2, _LANES), jnp.float32),
    mesh=_mesh(),
    scratch_types=[
        pltpu.VMEM((4, _D, _LANES), jnp.float32),
        pltpu.VMEM((4, _D, _LANES), jnp.float32),
    ]
    + [pltpu.SemaphoreType.DMA] * 8,
    compiler_params=pltpu.CompilerParams(
        use_tc_tiling_on_sc=True,
        needs_layout_passes=False,
        disable_bounds_checks=True,
    ),
    name="emb_relayout",
)
def _relayout(tt_hbm, out_hbm, in_v, tp_v, *sems):
    isems = sems[:4]
    osems = sems[4:]
    wid = lax.axis_index("s") * _NUM_CORES + lax.axis_index("c")
    ncols = _NCOLS_MAIN + 1
    n_extra = ncols - 244 * _NW
    tail_wid = (ncols - 1) % _NW
    nc = 244 + jnp.where(wid < n_extra, 1, 0)

    def col(t):
        return wid + _NW * t

    def start_in(t, b):
        pltpu.async_copy(
            tt_hbm.at[:, pl.ds(col(t) * _LANES, _LANES)], in_v.at[b], isems[b]
        )

    def wait_in(b):
        pltpu.make_async_copy(
            tt_hbm.at[:, pl.ds(0, _LANES)], in_v.at[b], isems[b]
        ).wait()

    def start_out(t, b):
        c = col(t)

        @pl.when(c != _NCOLS_MAIN)
        def _():
            pltpu.async_copy(
                tp_v.at[b], out_hbm.at[pl.ds(c * (_LANES // 2), _D)], osems[b]
            )

        @pl.when(c == _NCOLS_MAIN)
        def _():
            pltpu.async_copy(
                tp_v.at[b, pl.ds(0, _D // 2)],
                out_hbm.at[pl.ds(c * (_LANES // 2), _D // 2)],
                osems[b],
            )

    def wait_out(b):
        pltpu.make_async_copy(
            tt_hbm.at[pl.ds(0, _D), pl.ds(0, _LANES)], tp_v.at[b], osems[b]
        ).wait()

    def wait_out_half(b):
        pltpu.make_async_copy(
            tt_hbm.at[pl.ds(0, _D // 2), pl.ds(0, _LANES)],
            tp_v.at[b, pl.ds(0, _D // 2)],
            osems[b],
        ).wait()

    for b in range(3):

        @pl.when(b < nc)
        def _(b=b):
            start_in(b, b)

    @pl.loop(0, (nc + 3) // 4 * 4, step=4)
    def _(t0):
        for b in range(4):
            t = t0 + b

            @pl.when(t < nc)
            def _():
                @pl.when(t + 3 < nc)
                def _():
                    start_in(t + 3, (b + 3) % 4)

                wait_in(b)

                @pl.when(t >= 4)
                def _():
                    wait_out(b)

                _transpose_block(in_v.at[b], tp_v.at[b])
                start_out(t, b)

    @pl.when(wid == tail_wid)
    def _():
        wait_out_half(0)
        wait_out(1)
        wait_out(2)
        wait_out(3)

    @pl.when(wid != tail_wid)
    def _():
        wait_out(0)
        wait_out(1)
        wait_out(2)
        wait_out(3)


@functools.lru_cache(maxsize=None)
def _build_gather(B, V, D):
    assert B % _NW == 0
    b_per_w = B // _NW
    assert b_per_w % _CHUNK == 0
    n_chunks = b_per_w // _CHUNK
    assert n_chunks % _NBUF == 0

    @functools.partial(
        pl.kernel,
        out_type=jax.ShapeDtypeStruct((B, D), jnp.float32),
        mesh=_mesh(),
        scratch_types=[
            pltpu.VMEM((b_per_w,), jnp.int32),
            pltpu.VMEM((_NBUF, _CHUNK, D), jnp.float32),
        ]
        + [pltpu.SemaphoreType.DMA] * _NBUF,
        compiler_params=pltpu.CompilerParams(use_tc_tiling_on_sc=False),
        name="emb_gather",
    )
    def emb_gather(table_hbm, idx_hbm, out_hbm, idx_v, rows_v, *sems):
        wid = lax.axis_index("s") * _NUM_CORES + lax.axis_index("c")
        base = wid * b_per_w

        pltpu.sync_copy(idx_hbm.at[pl.ds(base, b_per_w)], idx_v)

        def start(g, b):
            pltpu.async_copy(
                table_hbm.at[idx_v.at[pl.ds(g * _CHUNK, _CHUNK)]],
                rows_v.at[b],
                sems[b],
            )

        def wait(b):
            pltpu.make_async_copy(
                table_hbm.at[pl.ds(0, _CHUNK)], rows_v.at[b], sems[b]
            ).wait()

        def writeout(g, b):
            pltpu.sync_copy(
                rows_v.at[b], out_hbm.at[pl.ds(base + g * _CHUNK, _CHUNK)]
            )

        for b in range(_NBUF - 1):
            start(b, b)

        @pl.loop(0, n_chunks, step=_NBUF)
        def _(g):
            for b in range(_NBUF):
                nxt = g + b + _NBUF - 1

                @pl.when(nxt < n_chunks)
                def _():
                    start(nxt, (b + _NBUF - 1) % _NBUF)

                wait(b)
                writeout(g + b, b)

    return emb_gather


def kernel(inputs, embeddings):
    V, D = embeddings.shape
    B = inputs.shape[0] * inputs.shape[1]
    idx = inputs.reshape(-1).astype(jnp.int32)
    linear = _relayout(embeddings.T).reshape(V, D)
    out = _build_gather(B, V, D)(linear, idx)
    return out.reshape(inputs.shape + (D,))


# speedup vs baseline: 2.1557x; 1.3370x over previous
import functools

import jax
import jax.numpy as jnp
from jax import lax
from jax.experimental import pallas as pl
from jax.experimental.pallas import tpu as pltpu
from jax.experimental.pallas import tpu_sc as plsc

_NUM_CORES = 2
_NUM_SUBCORES = 16
_NW = _NUM_CORES * _NUM_SUBCORES

_CHUNK = 256
_NBUF = 5


@functools.lru_cache(maxsize=None)
def _build_gather(B, V, D):
    assert B % _NW == 0
    b_per_w = B // _NW
    assert b_per_w % _CHUNK == 0
    n_chunks = b_per_w // _CHUNK
    assert n_chunks % _NBUF == 0

    mesh = plsc.VectorSubcoreMesh(
        core_axis_name="c",
        subcore_axis_name="s",
        num_cores=_NUM_CORES,
        num_subcores=_NUM_SUBCORES,
    )

    @functools.partial(
        pl.kernel,
        out_type=jax.ShapeDtypeStruct((B, D), jnp.float32),
        mesh=mesh,
        scratch_types=[
            pltpu.VMEM((b_per_w,), jnp.int32),
            pltpu.VMEM((_NBUF, _CHUNK, D), jnp.float32),
        ]
        + [pltpu.SemaphoreType.DMA] * _NBUF,
        compiler_params=pltpu.CompilerParams(use_tc_tiling_on_sc=False),
        name="emb_gather",
    )
    def emb_gather(table_hbm, idx_hbm, out_hbm, idx_v, rows_v, *sems):
        wid = lax.axis_index("s") * _NUM_CORES + lax.axis_index("c")
        base = wid * b_per_w

        pltpu.sync_copy(idx_hbm.at[pl.ds(base, b_per_w)], idx_v)

        def start(g, b):
            pltpu.async_copy(
                table_hbm.at[idx_v.at[pl.ds(g * _CHUNK, _CHUNK)]],
                rows_v.at[b],
                sems[b],
            )

        def wait(b):
            pltpu.make_async_copy(
                table_hbm.at[pl.ds(0, _CHUNK)], rows_v.at[b], sems[b]
            ).wait()

        def writeout(g, b):
            pltpu.sync_copy(
                rows_v.at[b], out_hbm.at[pl.ds(base + g * _CHUNK, _CHUNK)]
            )

        for b in range(_NBUF - 1):
            start(b, b)

        @pl.loop(0, n_chunks, step=_NBUF)
        def _(g):
            for b in range(_NBUF):
                nxt = g + b + _NBUF - 1

                @pl.when(nxt < n_chunks)
                def _():
                    start(nxt, (b + _NBUF - 1) % _NBUF)

                wait(b)
                writeout(g + b, b)

    return emb_gather


def kernel(inputs, embeddings):
    V, D = embeddings.shape
    B = inputs.shape[0] * inputs.shape[1]
    idx = inputs.reshape(-1).astype(jnp.int32)
    out = _build_gather(B, V, D)(embeddings, idx)
    return out.reshape(inputs.shape + (D,))
